# baseline (speedup 1.0000x reference)
import functools

import jax
import jax.numpy as jnp
from jax import lax
from jax.experimental import pallas as pl
from jax.experimental.pallas import tpu as pltpu

N = 1_000_000
H = 32
B = 131072
GRID = (N + B - 1) // B


def _tc_body(ap_ref, delay_ref, hist_ref, out_ref):
    d = delay_ref[...]
    hrow = lax.broadcasted_iota(jnp.int32, (H, B), 0)
    cmp = hrow == (d - 1)[None, :]
    masked = jnp.where(cmp, hist_ref[...], 0.0)
    red = jnp.sum(masked, axis=0)
    out_ref[...] = jnp.where(d == 0, ap_ref[...], red)


@jax.jit
def _tc_axon(ap, hist, delay):
    return pl.pallas_call(
        _tc_body,
        out_shape=jax.ShapeDtypeStruct((N,), jnp.float32),
        grid=(GRID,),
        in_specs=[
            pl.BlockSpec((B,), lambda i: (i,)),
            pl.BlockSpec((B,), lambda i: (i,)),
            pl.BlockSpec((H, B), lambda i: (0, i)),
        ],
        out_specs=pl.BlockSpec((B,), lambda i: (i,)),
    )(ap, delay, hist)


def kernel(action_potential, history, delay):
    return _tc_axon(action_potential, history, delay.astype(jnp.int32))
